# Initial kernel scaffold; baseline (speedup 1.0000x reference)
import jax
import jax.numpy as jnp
from jax.experimental import pallas as pl


def kernel(xyz, features, Wi, bi, Wo, bo, Wm, bm, Wk1, bk1, Wk2, bk2, Wc, bc, gamma, beta):
    raise NotImplementedError("write your pallas kernel here")


# speedup vs baseline: 4.8816x; 4.8816x over previous
import functools
import math

import jax
import jax.numpy as jnp
from jax import lax
from jax.experimental import pallas as pl
from jax.experimental.pallas import tpu as pltpu
from jax.experimental.pallas import tpu_sc as plsc

B, N, C, K, H = 4, 2048, 512, 16, 4
DH = C // H
NB = 8
BN = N // NB
NW = 32
PPW = (B * N) // NW


def _prep_body(f_ref, xyzp_ref, xyzt_ref, Wi_ref, bi_ref, Wk1_ref, bk1_ref,
               q_ref, k_ref, v_ref, h_ref, idx_ref):
    b = pl.program_id(0)
    f = f_ref[0]

    qkv = jnp.dot(f, Wi_ref[...].T, preferred_element_type=jnp.float32)
    qkv = qkv + bi_ref[...]
    q_ref[0] = qkv[:, :C]
    k_ref[0] = qkv[:, C:2 * C]
    v_ref[0] = qkv[:, 2 * C:]

    h_ref[...] = (jnp.dot(f, Wk1_ref[...].T, preferred_element_type=jnp.float32)
                  + bk1_ref[...])

    xb = xyzp_ref[0]
    xf = xyzt_ref[0]
    xc0, xc1, xc2 = xb[:, 0:1], xb[:, 1:2], xb[:, 2:3]
    xr0, xr1, xr2 = xf[0:1, :], xf[1:2, :], xf[2:3, :]
    dot = xc0 * xr0 + xc1 * xr1 + xc2 * xr2
    sqb = xc0 * xc0 + xc1 * xc1 + xc2 * xc2
    sqf = xr0 * xr0 + xr1 * xr1 + xr2 * xr2
    d2 = sqb + sqf - 2.0 * dot

    iota = lax.broadcasted_iota(jnp.int32, (BN, N), 1)
    col = lax.broadcasted_iota(jnp.int32, (BN, 128), 1)
    acc = jnp.zeros((BN, 128), jnp.int32)
    work = d2
    for kk in range(K):
        m = jnp.min(work, axis=1, keepdims=True)
        am = jnp.min(jnp.where(work == m, iota, N), axis=1, keepdims=True)
        acc = jnp.where(col == kk, am + b * N, acc)
        work = jnp.where(iota == am, jnp.float32(jnp.inf), work)
    idx_ref[...] = acc


def _run_prep(features, xyzp, xyzt, Wi, bi, Wk1, bk1):
    f32 = jnp.float32
    return pl.pallas_call(
        _prep_body,
        grid=(B, NB),
        in_specs=[
            pl.BlockSpec((1, BN, C), lambda b, nb: (b, nb, 0)),
            pl.BlockSpec((1, BN, 128), lambda b, nb: (b, nb, 0)),
            pl.BlockSpec((1, 3, N), lambda b, nb: (b, 0, 0)),
            pl.BlockSpec((3 * C, C), lambda b, nb: (0, 0)),
            pl.BlockSpec((1, 3 * C), lambda b, nb: (0, 0)),
            pl.BlockSpec((C, C), lambda b, nb: (0, 0)),
            pl.BlockSpec((1, C), lambda b, nb: (0, 0)),
        ],
        out_specs=[
            pl.BlockSpec((1, BN, C), lambda b, nb: (b, nb, 0)),
            pl.BlockSpec((1, BN, C), lambda b, nb: (b, nb, 0)),
            pl.BlockSpec((1, BN, C), lambda b, nb: (b, nb, 0)),
            pl.BlockSpec((BN, C), lambda b, nb: (b * NB + nb, 0)),
            pl.BlockSpec((BN, 128), lambda b, nb: (b * NB + nb, 0)),
        ],
        out_shape=[
            jax.ShapeDtypeStruct((B, N, C), f32),
            jax.ShapeDtypeStruct((B, N, C), f32),
            jax.ShapeDtypeStruct((B, N, C), f32),
            jax.ShapeDtypeStruct((B * N, C), f32),
            jax.ShapeDtypeStruct((B * N, 128), jnp.int32),
        ],
    )(features, xyzp, xyzt, Wi, bi, Wk1, bk1)


def _sc_body(h_hbm, idx_hbm, out_hbm, idxv, rows, outv, sem_g):
    c = lax.axis_index("c")
    s = lax.axis_index("s")
    wid = s * 2 + c
    base = wid * PPW
    pltpu.sync_copy(idx_hbm.at[pl.ds(base, PPW)], idxv)

    def body(p, carry):
        pltpu.async_copy(h_hbm.at[idxv.at[p, pl.ds(0, K)]], rows, sem_g).wait()
        for cc in range(C // 16):
            acc = rows[0, pl.ds(cc * 16, 16)]
            for r in range(1, K):
                acc = jnp.maximum(acc, rows[r, pl.ds(cc * 16, 16)])
            outv[pl.ds(cc * 16, 16)] = acc
        pltpu.sync_copy(outv, out_hbm.at[base + p])
        return carry

    lax.fori_loop(0, PPW, body, 0)


def _run_sc_gather_max(h_flat, idx_flat):
    mesh = plsc.VectorSubcoreMesh(core_axis_name="c", subcore_axis_name="s")
    fn = functools.partial(
        pl.kernel,
        mesh=mesh,
        out_type=jax.ShapeDtypeStruct((B * N, C), jnp.float32),
        scratch_types=[
            pltpu.VMEM((PPW, 128), jnp.int32),
            pltpu.VMEM((K, C), jnp.float32),
            pltpu.VMEM((C,), jnp.float32),
            pltpu.SemaphoreType.DMA,
        ],
    )(_sc_body)
    return fn(h_flat, idx_flat)


def _attn_body(q_ref, k_ref, v_ref, loc_ref, f_ref, Wo_ref, bo_ref, Wm_ref,
               bm_ref, Wk2_ref, bk2_ref, Wc_ref, bc_ref, g_ref, be_ref,
               out_ref):
    q = q_ref[0]
    kf = k_ref[0]
    vf = v_ref[0]
    scale = 1.0 / math.sqrt(DH)
    parts = []
    for hh in range(H):
        sl = slice(hh * DH, (hh + 1) * DH)
        s = jnp.dot(q[:, sl], kf[:, sl].T,
                    preferred_element_type=jnp.float32) * scale
        s = s - jnp.max(s, axis=1, keepdims=True)
        p = jnp.exp(s)
        p = p / jnp.sum(p, axis=1, keepdims=True)
        parts.append(jnp.dot(p, vf[:, sl], preferred_element_type=jnp.float32))
    ao = jnp.concatenate(parts, axis=1)

    attn_out = jnp.dot(ao, Wo_ref[...].T, preferred_element_type=jnp.float32) + bo_ref[...]
    glob = jnp.dot(attn_out, Wm_ref[...].T, preferred_element_type=jnp.float32) + bm_ref[...]

    loc = jnp.maximum(loc_ref[...], 0.0)
    loc = jnp.dot(loc, Wk2_ref[...].T, preferred_element_type=jnp.float32) + bk2_ref[...]

    Wc = Wc_ref[...]
    fused = (jnp.dot(glob, Wc[:, :C].T, preferred_element_type=jnp.float32)
             + jnp.dot(loc, Wc[:, C:].T, preferred_element_type=jnp.float32)
             + bc_ref[...])
    fused = jnp.maximum(fused, 0.0)

    x = fused + f_ref[0]
    mu = jnp.mean(x, axis=1, keepdims=True)
    var = jnp.mean((x - mu) * (x - mu), axis=1, keepdims=True)
    out_ref[0] = (x - mu) / jnp.sqrt(var + 1e-5) * g_ref[...] + be_ref[...]


def _run_attn_tail(q, k, v, local_flat, features, Wo, bo, Wm, bm, Wk2, bk2,
                   Wc, bc, gamma, beta):
    f32 = jnp.float32
    full = lambda r, c: pl.BlockSpec((r, c), lambda b, nb: (0, 0))
    return pl.pallas_call(
        _attn_body,
        grid=(B, NB),
        in_specs=[
            pl.BlockSpec((1, BN, C), lambda b, nb: (b, nb, 0)),
            pl.BlockSpec((1, N, C), lambda b, nb: (b, 0, 0)),
            pl.BlockSpec((1, N, C), lambda b, nb: (b, 0, 0)),
            pl.BlockSpec((BN, C), lambda b, nb: (b * NB + nb, 0)),
            pl.BlockSpec((1, BN, C), lambda b, nb: (b, nb, 0)),
            full(C, C), full(1, C),
            full(C, C), full(1, C),
            full(C, C), full(1, C),
            full(C, 2 * C), full(1, C),
            full(1, C), full(1, C),
        ],
        out_specs=pl.BlockSpec((1, BN, C), lambda b, nb: (b, nb, 0)),
        out_shape=jax.ShapeDtypeStruct((B, N, C), f32),
    )(q, k, v, local_flat, features, Wo, bo, Wm, bm, Wk2, bk2, Wc, bc,
      gamma, beta)


def kernel(xyz, features, Wi, bi, Wo, bo, Wm, bm, Wk1, bk1, Wk2, bk2, Wc, bc,
           gamma, beta):
    f32 = jnp.float32
    xyzp = jnp.pad(xyz, ((0, 0), (0, 0), (0, 125))).astype(f32)
    xyzt = jnp.transpose(xyz, (0, 2, 1)).astype(f32)
    r2 = lambda t: t.reshape(1, -1)

    q, k, v, h_flat, idx_flat = _run_prep(
        features, xyzp, xyzt, Wi, r2(bi), Wk1, r2(bk1))

    local_flat = _run_sc_gather_max(h_flat, idx_flat)

    return _run_attn_tail(q, k, v, local_flat, features, Wo, r2(bo), Wm,
                          r2(bm), Wk2, r2(bk2), Wc, r2(bc), r2(gamma),
                          r2(beta))


# speedup vs baseline: 4.8906x; 1.0018x over previous
import functools
import math

import jax
import jax.numpy as jnp
from jax import lax
from jax.experimental import pallas as pl
from jax.experimental.pallas import tpu as pltpu
from jax.experimental.pallas import tpu_sc as plsc

B, N, C, K, H = 4, 2048, 512, 16, 4
DH = C // H
NB = 8
BN = N // NB
NW = 32
PPW = (B * N) // NW


def _prep_body(f_ref, xyzp_ref, xyzt_ref, Wi_ref, bi_ref, Wk1_ref, bk1_ref,
               q_ref, k_ref, v_ref, h_ref, idx_ref):
    b = pl.program_id(0)
    f = f_ref[0]

    qkv = jnp.dot(f, Wi_ref[...].T, preferred_element_type=jnp.float32)
    qkv = qkv + bi_ref[...]
    q_ref[0] = qkv[:, :C]
    k_ref[0] = qkv[:, C:2 * C]
    v_ref[0] = qkv[:, 2 * C:]

    h_ref[...] = (jnp.dot(f, Wk1_ref[...].T, preferred_element_type=jnp.float32)
                  + bk1_ref[...])

    xb = xyzp_ref[0]
    xf = xyzt_ref[0]
    xc0, xc1, xc2 = xb[:, 0:1], xb[:, 1:2], xb[:, 2:3]
    xr0, xr1, xr2 = xf[0:1, :], xf[1:2, :], xf[2:3, :]
    dot = jnp.dot(xb, xf, preferred_element_type=jnp.float32)
    sqb = xc0 * xc0 + xc1 * xc1 + xc2 * xc2
    sqf = xr0 * xr0 + xr1 * xr1 + xr2 * xr2
    d2 = sqb + sqf - 2.0 * dot

    iota = lax.broadcasted_iota(jnp.int32, (BN, N), 1)
    col = lax.broadcasted_iota(jnp.int32, (BN, 128), 1)
    acc = jnp.zeros((BN, 128), jnp.int32)
    work = d2
    for kk in range(K):
        m = jnp.min(work, axis=1, keepdims=True)
        am = jnp.min(jnp.where(work == m, iota, N), axis=1, keepdims=True)
        acc = jnp.where(col == kk, am + b * N, acc)
        work = jnp.where(iota == am, jnp.float32(jnp.inf), work)
    idx_ref[...] = acc


def _run_prep(features, xyzp, xyzt, Wi, bi, Wk1, bk1):
    f32 = jnp.float32
    return pl.pallas_call(
        _prep_body,
        grid=(B, NB),
        in_specs=[
            pl.BlockSpec((1, BN, C), lambda b, nb: (b, nb, 0)),
            pl.BlockSpec((1, BN, 128), lambda b, nb: (b, nb, 0)),
            pl.BlockSpec((1, 128, N), lambda b, nb: (b, 0, 0)),
            pl.BlockSpec((3 * C, C), lambda b, nb: (0, 0)),
            pl.BlockSpec((1, 3 * C), lambda b, nb: (0, 0)),
            pl.BlockSpec((C, C), lambda b, nb: (0, 0)),
            pl.BlockSpec((1, C), lambda b, nb: (0, 0)),
        ],
        out_specs=[
            pl.BlockSpec((1, BN, C), lambda b, nb: (b, nb, 0)),
            pl.BlockSpec((1, BN, C), lambda b, nb: (b, nb, 0)),
            pl.BlockSpec((1, BN, C), lambda b, nb: (b, nb, 0)),
            pl.BlockSpec((BN, C), lambda b, nb: (b * NB + nb, 0)),
            pl.BlockSpec((BN, 128), lambda b, nb: (b * NB + nb, 0)),
        ],
        out_shape=[
            jax.ShapeDtypeStruct((B, N, C), f32),
            jax.ShapeDtypeStruct((B, N, C), f32),
            jax.ShapeDtypeStruct((B, N, C), f32),
            jax.ShapeDtypeStruct((B * N, C), f32),
            jax.ShapeDtypeStruct((B * N, 128), jnp.int32),
        ],
    )(features, xyzp, xyzt, Wi, bi, Wk1, bk1)


def _sc_body(h_hbm, idx_hbm, out_hbm, idxv, rows, outv, sem_g):
    c = lax.axis_index("c")
    s = lax.axis_index("s")
    wid = s * 2 + c
    base = wid * PPW
    pltpu.sync_copy(idx_hbm.at[pl.ds(base, PPW)], idxv)

    def body(p, carry):
        pltpu.async_copy(h_hbm.at[idxv.at[p, pl.ds(0, K)]], rows, sem_g).wait()
        for cc in range(C // 16):
            acc = rows[0, pl.ds(cc * 16, 16)]
            for r in range(1, K):
                acc = jnp.maximum(acc, rows[r, pl.ds(cc * 16, 16)])
            outv[pl.ds(cc * 16, 16)] = acc
        pltpu.sync_copy(outv, out_hbm.at[base + p])
        return carry

    lax.fori_loop(0, PPW, body, 0)


def _run_sc_gather_max(h_flat, idx_flat):
    mesh = plsc.VectorSubcoreMesh(core_axis_name="c", subcore_axis_name="s")
    fn = functools.partial(
        pl.kernel,
        mesh=mesh,
        out_type=jax.ShapeDtypeStruct((B * N, C), jnp.float32),
        scratch_types=[
            pltpu.VMEM((PPW, 128), jnp.int32),
            pltpu.VMEM((K, C), jnp.float32),
            pltpu.VMEM((C,), jnp.float32),
            pltpu.SemaphoreType.DMA,
        ],
    )(_sc_body)
    return fn(h_flat, idx_flat)


def _attn_body(q_ref, k_ref, v_ref, loc_ref, f_ref, Wo_ref, bo_ref, Wm_ref,
               bm_ref, Wk2_ref, bk2_ref, Wc_ref, bc_ref, g_ref, be_ref,
               out_ref):
    q = q_ref[0]
    kf = k_ref[0]
    vf = v_ref[0]
    scale = 1.0 / math.sqrt(DH)
    parts = []
    for hh in range(H):
        sl = slice(hh * DH, (hh + 1) * DH)
        s = jnp.dot(q[:, sl], kf[:, sl].T,
                    preferred_element_type=jnp.float32) * scale
        s = s - jnp.max(s, axis=1, keepdims=True)
        p = jnp.exp(s)
        p = p / jnp.sum(p, axis=1, keepdims=True)
        parts.append(jnp.dot(p, vf[:, sl], preferred_element_type=jnp.float32))
    ao = jnp.concatenate(parts, axis=1)

    attn_out = jnp.dot(ao, Wo_ref[...].T, preferred_element_type=jnp.float32) + bo_ref[...]
    glob = jnp.dot(attn_out, Wm_ref[...].T, preferred_element_type=jnp.float32) + bm_ref[...]

    loc = jnp.maximum(loc_ref[...], 0.0)
    loc = jnp.dot(loc, Wk2_ref[...].T, preferred_element_type=jnp.float32) + bk2_ref[...]

    Wc = Wc_ref[...]
    fused = (jnp.dot(glob, Wc[:, :C].T, preferred_element_type=jnp.float32)
             + jnp.dot(loc, Wc[:, C:].T, preferred_element_type=jnp.float32)
             + bc_ref[...])
    fused = jnp.maximum(fused, 0.0)

    x = fused + f_ref[0]
    mu = jnp.mean(x, axis=1, keepdims=True)
    var = jnp.mean((x - mu) * (x - mu), axis=1, keepdims=True)
    out_ref[0] = (x - mu) / jnp.sqrt(var + 1e-5) * g_ref[...] + be_ref[...]


def _run_attn_tail(q, k, v, local_flat, features, Wo, bo, Wm, bm, Wk2, bk2,
                   Wc, bc, gamma, beta):
    f32 = jnp.float32
    full = lambda r, c: pl.BlockSpec((r, c), lambda b, nb: (0, 0))
    return pl.pallas_call(
        _attn_body,
        grid=(B, NB),
        in_specs=[
            pl.BlockSpec((1, BN, C), lambda b, nb: (b, nb, 0)),
            pl.BlockSpec((1, N, C), lambda b, nb: (b, 0, 0)),
            pl.BlockSpec((1, N, C), lambda b, nb: (b, 0, 0)),
            pl.BlockSpec((BN, C), lambda b, nb: (b * NB + nb, 0)),
            pl.BlockSpec((1, BN, C), lambda b, nb: (b, nb, 0)),
            full(C, C), full(1, C),
            full(C, C), full(1, C),
            full(C, C), full(1, C),
            full(C, 2 * C), full(1, C),
            full(1, C), full(1, C),
        ],
        out_specs=pl.BlockSpec((1, BN, C), lambda b, nb: (b, nb, 0)),
        out_shape=jax.ShapeDtypeStruct((B, N, C), f32),
    )(q, k, v, local_flat, features, Wo, bo, Wm, bm, Wk2, bk2, Wc, bc,
      gamma, beta)


def kernel(xyz, features, Wi, bi, Wo, bo, Wm, bm, Wk1, bk1, Wk2, bk2, Wc, bc,
           gamma, beta):
    f32 = jnp.float32
    xyzp = jnp.pad(xyz, ((0, 0), (0, 0), (0, 125))).astype(f32)
    xyzt = jnp.pad(jnp.transpose(xyz, (0, 2, 1)),
                   ((0, 0), (0, 125), (0, 0))).astype(f32)
    r2 = lambda t: t.reshape(1, -1)

    q, k, v, h_flat, idx_flat = _run_prep(
        features, xyzp, xyzt, Wi, r2(bi), Wk1, r2(bk1))

    local_flat = _run_sc_gather_max(h_flat, idx_flat)

    return _run_attn_tail(q, k, v, local_flat, features, Wo, r2(bo), Wm,
                          r2(bm), Wk2, r2(bk2), Wc, r2(bc), r2(gamma),
                          r2(beta))


# speedup vs baseline: 8.1642x; 1.6694x over previous
import functools
import math

import jax
import jax.numpy as jnp
from jax import lax
from jax.experimental import pallas as pl
from jax.experimental.pallas import tpu as pltpu
from jax.experimental.pallas import tpu_sc as plsc

B, N, C, K, H = 4, 2048, 512, 16, 4
DH = C // H
NB = 8
BN = N // NB
NW = 32
PPW = (B * N) // NW


def _prep_body(f_ref, xyzp_ref, xyzt_ref, Wi_ref, bi_ref, Wk1_ref, bk1_ref,
               q_ref, k_ref, v_ref, h_ref, idx_ref):
    b = pl.program_id(0)
    f = f_ref[0]

    qkv = jnp.dot(f, Wi_ref[...].T, preferred_element_type=jnp.float32)
    qkv = qkv + bi_ref[...]
    q_ref[0] = qkv[:, :C]
    k_ref[0] = qkv[:, C:2 * C]
    v_ref[0] = qkv[:, 2 * C:]

    h_ref[...] = (jnp.dot(f, Wk1_ref[...].T, preferred_element_type=jnp.float32)
                  + bk1_ref[...])

    xb = xyzp_ref[0]
    xf = xyzt_ref[0]
    xc0, xc1, xc2 = xb[:, 0:1], xb[:, 1:2], xb[:, 2:3]
    xr0, xr1, xr2 = xf[0:1, :], xf[1:2, :], xf[2:3, :]
    dot = jnp.dot(xb, xf, preferred_element_type=jnp.float32)
    sqb = xc0 * xc0 + xc1 * xc1 + xc2 * xc2
    sqf = xr0 * xr0 + xr1 * xr1 + xr2 * xr2
    d2 = sqb + sqf - 2.0 * dot

    iota = lax.broadcasted_iota(jnp.int32, (BN, N), 1)
    col = lax.broadcasted_iota(jnp.int32, (BN, 128), 1)
    acc = jnp.zeros((BN, 128), jnp.int32)
    work = d2
    for kk in range(K):
        m = jnp.min(work, axis=1, keepdims=True)
        am = jnp.min(jnp.where(work == m, iota, N), axis=1, keepdims=True)
        acc = jnp.where(col == kk, am + b * N, acc)
        work = jnp.where(iota == am, jnp.float32(jnp.inf), work)
    idx_ref[...] = acc


def _run_prep(features, xyzp, xyzt, Wi, bi, Wk1, bk1):
    f32 = jnp.float32
    return pl.pallas_call(
        _prep_body,
        grid=(B, NB),
        in_specs=[
            pl.BlockSpec((1, BN, C), lambda b, nb: (b, nb, 0)),
            pl.BlockSpec((1, BN, 128), lambda b, nb: (b, nb, 0)),
            pl.BlockSpec((1, 128, N), lambda b, nb: (b, 0, 0)),
            pl.BlockSpec((3 * C, C), lambda b, nb: (0, 0)),
            pl.BlockSpec((1, 3 * C), lambda b, nb: (0, 0)),
            pl.BlockSpec((C, C), lambda b, nb: (0, 0)),
            pl.BlockSpec((1, C), lambda b, nb: (0, 0)),
        ],
        out_specs=[
            pl.BlockSpec((1, BN, C), lambda b, nb: (b, nb, 0)),
            pl.BlockSpec((1, BN, C), lambda b, nb: (b, nb, 0)),
            pl.BlockSpec((1, BN, C), lambda b, nb: (b, nb, 0)),
            pl.BlockSpec((BN, C), lambda b, nb: (b * NB + nb, 0)),
            pl.BlockSpec((BN, 128), lambda b, nb: (b * NB + nb, 0)),
        ],
        out_shape=[
            jax.ShapeDtypeStruct((B, N, C), f32),
            jax.ShapeDtypeStruct((B, N, C), f32),
            jax.ShapeDtypeStruct((B, N, C), f32),
            jax.ShapeDtypeStruct((B * N, C), f32),
            jax.ShapeDtypeStruct((B * N, 128), jnp.int32),
        ],
    )(features, xyzp, xyzt, Wi, bi, Wk1, bk1)


_OB = 64


def _sc_body(h_hbm, idx_hbm, out_hbm, idxv, rows0, rows1, outs, sem0, sem1):
    c = lax.axis_index("c")
    s = lax.axis_index("s")
    wid = s * 2 + c
    base = wid * PPW
    pltpu.sync_copy(idx_hbm.at[pl.ds(base, PPW)], idxv)

    def fire(p, rbuf, sem):
        pltpu.async_copy(h_hbm.at[idxv.at[p, pl.ds(0, K)]], rbuf, sem)

    def wait(p, rbuf, sem):
        pltpu.make_async_copy(h_hbm.at[idxv.at[p, pl.ds(0, K)]], rbuf, sem).wait()

    def reduce_into(rbuf, orow):
        for cc in range(C // 16):
            acc = rbuf[0, pl.ds(cc * 16, 16)]
            for r in range(1, K):
                acc = jnp.maximum(acc, rbuf[r, pl.ds(cc * 16, 16)])
            outs[orow, pl.ds(cc * 16, 16)] = acc

    fire(0, rows0, sem0)
    for ob in range(PPW // _OB):
        def pair(i, carry):
            p0 = ob * _OB + 2 * i
            p1 = p0 + 1
            pn = jnp.minimum(p1 + 1, PPW - 1)
            fire(p1, rows1, sem1)
            wait(p0, rows0, sem0)
            reduce_into(rows0, 2 * i)
            fire(pn, rows0, sem0)
            wait(p1, rows1, sem1)
            reduce_into(rows1, 2 * i + 1)
            return carry

        lax.fori_loop(0, _OB // 2, pair, 0)
        pltpu.sync_copy(outs, out_hbm.at[pl.ds(base + ob * _OB, _OB)])
    wait(PPW - 1, rows0, sem0)


def _run_sc_gather_max(h_flat, idx_flat):
    mesh = plsc.VectorSubcoreMesh(core_axis_name="c", subcore_axis_name="s")
    fn = functools.partial(
        pl.kernel,
        mesh=mesh,
        out_type=jax.ShapeDtypeStruct((B * N, C), jnp.float32),
        scratch_types=[
            pltpu.VMEM((PPW, 128), jnp.int32),
            pltpu.VMEM((K, C), jnp.float32),
            pltpu.VMEM((K, C), jnp.float32),
            pltpu.VMEM((_OB, C), jnp.float32),
            pltpu.SemaphoreType.DMA,
            pltpu.SemaphoreType.DMA,
        ],
    )(_sc_body)
    return fn(h_flat, idx_flat)


def _attn_body(q_ref, k_ref, v_ref, loc_ref, f_ref, Wo_ref, bo_ref, Wm_ref,
               bm_ref, Wk2_ref, bk2_ref, Wc_ref, bc_ref, g_ref, be_ref,
               out_ref):
    q = q_ref[0]
    kf = k_ref[0]
    vf = v_ref[0]
    scale = 1.0 / math.sqrt(DH)
    parts = []
    for hh in range(H):
        sl = slice(hh * DH, (hh + 1) * DH)
        s = jnp.dot(q[:, sl], kf[:, sl].T,
                    preferred_element_type=jnp.float32) * scale
        s = s - jnp.max(s, axis=1, keepdims=True)
        p = jnp.exp(s)
        p = p / jnp.sum(p, axis=1, keepdims=True)
        parts.append(jnp.dot(p, vf[:, sl], preferred_element_type=jnp.float32))
    ao = jnp.concatenate(parts, axis=1)

    attn_out = jnp.dot(ao, Wo_ref[...].T, preferred_element_type=jnp.float32) + bo_ref[...]
    glob = jnp.dot(attn_out, Wm_ref[...].T, preferred_element_type=jnp.float32) + bm_ref[...]

    loc = jnp.maximum(loc_ref[...], 0.0)
    loc = jnp.dot(loc, Wk2_ref[...].T, preferred_element_type=jnp.float32) + bk2_ref[...]

    Wc = Wc_ref[...]
    fused = (jnp.dot(glob, Wc[:, :C].T, preferred_element_type=jnp.float32)
             + jnp.dot(loc, Wc[:, C:].T, preferred_element_type=jnp.float32)
             + bc_ref[...])
    fused = jnp.maximum(fused, 0.0)

    x = fused + f_ref[0]
    mu = jnp.mean(x, axis=1, keepdims=True)
    var = jnp.mean((x - mu) * (x - mu), axis=1, keepdims=True)
    out_ref[0] = (x - mu) / jnp.sqrt(var + 1e-5) * g_ref[...] + be_ref[...]


def _run_attn_tail(q, k, v, local_flat, features, Wo, bo, Wm, bm, Wk2, bk2,
                   Wc, bc, gamma, beta):
    f32 = jnp.float32
    full = lambda r, c: pl.BlockSpec((r, c), lambda b, nb: (0, 0))
    return pl.pallas_call(
        _attn_body,
        grid=(B, NB),
        in_specs=[
            pl.BlockSpec((1, BN, C), lambda b, nb: (b, nb, 0)),
            pl.BlockSpec((1, N, C), lambda b, nb: (b, 0, 0)),
            pl.BlockSpec((1, N, C), lambda b, nb: (b, 0, 0)),
            pl.BlockSpec((BN, C), lambda b, nb: (b * NB + nb, 0)),
            pl.BlockSpec((1, BN, C), lambda b, nb: (b, nb, 0)),
            full(C, C), full(1, C),
            full(C, C), full(1, C),
            full(C, C), full(1, C),
            full(C, 2 * C), full(1, C),
            full(1, C), full(1, C),
        ],
        out_specs=pl.BlockSpec((1, BN, C), lambda b, nb: (b, nb, 0)),
        out_shape=jax.ShapeDtypeStruct((B, N, C), f32),
    )(q, k, v, local_flat, features, Wo, bo, Wm, bm, Wk2, bk2, Wc, bc,
      gamma, beta)


def kernel(xyz, features, Wi, bi, Wo, bo, Wm, bm, Wk1, bk1, Wk2, bk2, Wc, bc,
           gamma, beta):
    f32 = jnp.float32
    xyzp = jnp.pad(xyz, ((0, 0), (0, 0), (0, 125))).astype(f32)
    xyzt = jnp.pad(jnp.transpose(xyz, (0, 2, 1)),
                   ((0, 0), (0, 125), (0, 0))).astype(f32)
    r2 = lambda t: t.reshape(1, -1)

    q, k, v, h_flat, idx_flat = _run_prep(
        features, xyzp, xyzt, Wi, r2(bi), Wk1, r2(bk1))

    local_flat = _run_sc_gather_max(h_flat, idx_flat)

    return _run_attn_tail(q, k, v, local_flat, features, Wo, r2(bo), Wm,
                          r2(bm), Wk2, r2(bk2), Wc, r2(bc), r2(gamma),
                          r2(beta))


# speedup vs baseline: 9.5628x; 1.1713x over previous
import functools
import math

import jax
import jax.numpy as jnp
from jax import lax
from jax.experimental import pallas as pl
from jax.experimental.pallas import tpu as pltpu
from jax.experimental.pallas import tpu_sc as plsc

B, N, C, K, H = 4, 2048, 512, 16, 4
DH = C // H
NB = 8
BN = N // NB
NW = 32
PPW = (B * N) // NW


def _prep_body(f_ref, xyzp_ref, xyzt_ref, Wi_ref, bi_ref, Wk1_ref, bk1_ref,
               q_ref, k_ref, v_ref, h_ref, idx_ref):
    b = pl.program_id(0)
    f = f_ref[0]

    qkv = jnp.dot(f, Wi_ref[...].T, preferred_element_type=jnp.float32)
    qkv = qkv + bi_ref[...]
    q_ref[0] = qkv[:, :C]
    k_ref[0] = qkv[:, C:2 * C]
    v_ref[0] = qkv[:, 2 * C:]

    h_ref[...] = (jnp.dot(f, Wk1_ref[...].T, preferred_element_type=jnp.float32)
                  + bk1_ref[...])

    xb = xyzp_ref[0]
    xf = xyzt_ref[0]
    xc0, xc1, xc2 = xb[:, 0:1], xb[:, 1:2], xb[:, 2:3]
    xr0, xr1, xr2 = xf[0:1, :], xf[1:2, :], xf[2:3, :]
    dot = jnp.dot(xb, xf, preferred_element_type=jnp.float32)
    sqb = xc0 * xc0 + xc1 * xc1 + xc2 * xc2
    sqf = xr0 * xr0 + xr1 * xr1 + xr2 * xr2
    d2 = sqb + sqf - 2.0 * dot

    iota = lax.broadcasted_iota(jnp.int32, (BN, N), 1)
    col = lax.broadcasted_iota(jnp.int32, (BN, 128), 1)
    acc = jnp.zeros((BN, 128), jnp.int32)
    work = d2
    for kk in range(K):
        m = jnp.min(work, axis=1, keepdims=True)
        am = jnp.min(jnp.where(work == m, iota, N), axis=1, keepdims=True)
        acc = jnp.where(col == kk, am + b * N, acc)
        work = jnp.where(iota == am, jnp.float32(jnp.inf), work)
    idx_ref[...] = acc


def _run_prep(features, xyzp, xyzt, Wi, bi, Wk1, bk1):
    f32 = jnp.float32
    return pl.pallas_call(
        _prep_body,
        grid=(B, NB),
        in_specs=[
            pl.BlockSpec((1, BN, C), lambda b, nb: (b, nb, 0)),
            pl.BlockSpec((1, BN, 128), lambda b, nb: (b, nb, 0)),
            pl.BlockSpec((1, 128, N), lambda b, nb: (b, 0, 0)),
            pl.BlockSpec((3 * C, C), lambda b, nb: (0, 0)),
            pl.BlockSpec((1, 3 * C), lambda b, nb: (0, 0)),
            pl.BlockSpec((C, C), lambda b, nb: (0, 0)),
            pl.BlockSpec((1, C), lambda b, nb: (0, 0)),
        ],
        out_specs=[
            pl.BlockSpec((1, BN, C), lambda b, nb: (b, nb, 0)),
            pl.BlockSpec((1, BN, C), lambda b, nb: (b, nb, 0)),
            pl.BlockSpec((1, BN, C), lambda b, nb: (b, nb, 0)),
            pl.BlockSpec((BN, C), lambda b, nb: (b * NB + nb, 0)),
            pl.BlockSpec((BN, 128), lambda b, nb: (b * NB + nb, 0)),
        ],
        out_shape=[
            jax.ShapeDtypeStruct((B, N, C), f32),
            jax.ShapeDtypeStruct((B, N, C), f32),
            jax.ShapeDtypeStruct((B, N, C), f32),
            jax.ShapeDtypeStruct((B * N, C), f32),
            jax.ShapeDtypeStruct((B * N, 128), jnp.int32),
        ],
    )(features, xyzp, xyzt, Wi, bi, Wk1, bk1)


_OB = 64


def _sc_body(h_hbm, idx_hbm, out_hbm, idxv, rows0, rows1, outs, sem0, sem1):
    c = lax.axis_index("c")
    s = lax.axis_index("s")
    wid = s * 2 + c
    base = wid * PPW
    pltpu.sync_copy(idx_hbm.at[pl.ds(base, PPW)], idxv)

    def fire(p, rbuf, sem):
        pltpu.async_copy(h_hbm.at[idxv.at[p, pl.ds(0, K)]], rbuf, sem)

    def wait(p, rbuf, sem):
        pltpu.make_async_copy(h_hbm.at[idxv.at[p, pl.ds(0, K)]], rbuf, sem).wait()

    def reduce_into(rbuf, orow):
        for cc in range(C // 16):
            acc = rbuf[0, pl.ds(cc * 16, 16)]
            for r in range(1, K):
                acc = jnp.maximum(acc, rbuf[r, pl.ds(cc * 16, 16)])
            outs[orow, pl.ds(cc * 16, 16)] = acc

    fire(0, rows0, sem0)
    for ob in range(PPW // _OB):
        def pair(i, carry):
            p0 = ob * _OB + 2 * i
            p1 = p0 + 1
            pn = jnp.minimum(p1 + 1, PPW - 1)
            fire(p1, rows1, sem1)
            wait(p0, rows0, sem0)
            reduce_into(rows0, 2 * i)
            fire(pn, rows0, sem0)
            wait(p1, rows1, sem1)
            reduce_into(rows1, 2 * i + 1)
            return carry

        lax.fori_loop(0, _OB // 2, pair, 0)
        pltpu.sync_copy(outs, out_hbm.at[pl.ds(base + ob * _OB, _OB)])
    wait(PPW - 1, rows0, sem0)


def _run_sc_gather_max(h_flat, idx_flat):
    mesh = plsc.VectorSubcoreMesh(core_axis_name="c", subcore_axis_name="s")
    fn = functools.partial(
        pl.kernel,
        mesh=mesh,
        out_type=jax.ShapeDtypeStruct((B * N, C), jnp.float32),
        scratch_types=[
            pltpu.VMEM((PPW, 128), jnp.int32),
            pltpu.VMEM((K, C), jnp.float32),
            pltpu.VMEM((K, C), jnp.float32),
            pltpu.VMEM((_OB, C), jnp.float32),
            pltpu.SemaphoreType.DMA,
            pltpu.SemaphoreType.DMA,
        ],
    )(_sc_body)
    return fn(h_flat, idx_flat)


def _attn_body(q_ref, k_ref, v_ref, Wo_ref, bo_ref, Wm_ref, bm_ref, glob_ref):
    q = q_ref[0]
    kf = k_ref[0]
    vf = v_ref[0]
    scale = 1.0 / math.sqrt(DH)
    parts = []
    for hh in range(H):
        sl = slice(hh * DH, (hh + 1) * DH)
        s = jnp.dot(q[:, sl], kf[:, sl].T,
                    preferred_element_type=jnp.float32) * scale
        s = s - jnp.max(s, axis=1, keepdims=True)
        p = jnp.exp(s)
        p = p / jnp.sum(p, axis=1, keepdims=True)
        parts.append(jnp.dot(p, vf[:, sl], preferred_element_type=jnp.float32))
    ao = jnp.concatenate(parts, axis=1)

    attn_out = jnp.dot(ao, Wo_ref[...].T, preferred_element_type=jnp.float32) + bo_ref[...]
    glob_ref[0] = (jnp.dot(attn_out, Wm_ref[...].T,
                           preferred_element_type=jnp.float32) + bm_ref[...])


def _run_attn(q, k, v, Wo, bo, Wm, bm):
    full = lambda r, c: pl.BlockSpec((r, c), lambda b, nb: (0, 0))
    return pl.pallas_call(
        _attn_body,
        grid=(B, NB),
        in_specs=[
            pl.BlockSpec((1, BN, C), lambda b, nb: (b, nb, 0)),
            pl.BlockSpec((1, N, C), lambda b, nb: (b, 0, 0)),
            pl.BlockSpec((1, N, C), lambda b, nb: (b, 0, 0)),
            full(C, C), full(1, C),
            full(C, C), full(1, C),
        ],
        out_specs=pl.BlockSpec((1, BN, C), lambda b, nb: (b, nb, 0)),
        out_shape=jax.ShapeDtypeStruct((B, N, C), jnp.float32),
    )(q, k, v, Wo, bo, Wm, bm)


def _tail_body(glob_ref, loc_ref, f_ref, Wk2_ref, bk2_ref, Wc_ref, bc_ref,
               g_ref, be_ref, out_ref):
    glob = glob_ref[0]
    loc = jnp.maximum(loc_ref[...], 0.0)
    loc = jnp.dot(loc, Wk2_ref[...].T, preferred_element_type=jnp.float32) + bk2_ref[...]

    Wc = Wc_ref[...]
    fused = (jnp.dot(glob, Wc[:, :C].T, preferred_element_type=jnp.float32)
             + jnp.dot(loc, Wc[:, C:].T, preferred_element_type=jnp.float32)
             + bc_ref[...])
    fused = jnp.maximum(fused, 0.0)

    x = fused + f_ref[0]
    mu = jnp.mean(x, axis=1, keepdims=True)
    var = jnp.mean((x - mu) * (x - mu), axis=1, keepdims=True)
    out_ref[0] = (x - mu) / jnp.sqrt(var + 1e-5) * g_ref[...] + be_ref[...]


def _run_tail(glob, local_flat, features, Wk2, bk2, Wc, bc, gamma, beta):
    full = lambda r, c: pl.BlockSpec((r, c), lambda b, nb: (0, 0))
    return pl.pallas_call(
        _tail_body,
        grid=(B, NB),
        in_specs=[
            pl.BlockSpec((1, BN, C), lambda b, nb: (b, nb, 0)),
            pl.BlockSpec((BN, C), lambda b, nb: (b * NB + nb, 0)),
            pl.BlockSpec((1, BN, C), lambda b, nb: (b, nb, 0)),
            full(C, C), full(1, C),
            full(C, 2 * C), full(1, C),
            full(1, C), full(1, C),
        ],
        out_specs=pl.BlockSpec((1, BN, C), lambda b, nb: (b, nb, 0)),
        out_shape=jax.ShapeDtypeStruct((B, N, C), jnp.float32),
    )(glob, local_flat, features, Wk2, bk2, Wc, bc, gamma, beta)


def kernel(xyz, features, Wi, bi, Wo, bo, Wm, bm, Wk1, bk1, Wk2, bk2, Wc, bc,
           gamma, beta):
    f32 = jnp.float32
    xyzp = jnp.pad(xyz, ((0, 0), (0, 0), (0, 125))).astype(f32)
    xyzt = jnp.pad(jnp.transpose(xyz, (0, 2, 1)),
                   ((0, 0), (0, 125), (0, 0))).astype(f32)
    r2 = lambda t: t.reshape(1, -1)

    q, k, v, h_flat, idx_flat = _run_prep(
        features, xyzp, xyzt, Wi, r2(bi), Wk1, r2(bk1))

    local_flat = _run_sc_gather_max(h_flat, idx_flat)
    glob = _run_attn(q, k, v, Wo, r2(bo), Wm, r2(bm))

    return _run_tail(glob, local_flat, features, Wk2, r2(bk2), Wc, r2(bc),
                     r2(gamma), r2(beta))
